# Initial kernel scaffold; baseline (speedup 1.0000x reference)
import jax
import jax.numpy as jnp
from jax.experimental import pallas as pl


def kernel(x, W, b):
    raise NotImplementedError("write your pallas kernel here")


# speedup vs baseline: 1.2046x; 1.2046x over previous
import functools

import jax
import jax.numpy as jnp
import numpy as np
from jax.experimental import pallas as pl
from jax.experimental.pallas import tpu as pltpu

_A = 100000
_TILE = 2048

_KS0 = np.uint32(0)
_KS1 = np.uint32(42)
_KS2 = np.uint32(_KS0 ^ _KS1 ^ np.uint32(0x1BD11BDA))
_ROT_A = (13, 15, 26, 6)
_ROT_B = (17, 29, 16, 24)
_TINY = np.float32(np.finfo(np.float32).tiny)


def _rotl(v, r):
    return jax.lax.shift_left(v, np.uint32(r)) | jax.lax.shift_right_logical(
        v, np.uint32(32 - r)
    )


def _gumbel_bits(idx_u32):
    ks = (_KS0, _KS1, _KS2)
    x0 = jnp.zeros_like(idx_u32) + _KS0
    x1 = idx_u32 + _KS1
    rots = (_ROT_A, _ROT_B)
    for i in range(5):
        for r in rots[i % 2]:
            x0 = x0 + x1
            x1 = _rotl(x1, r)
            x1 = x1 ^ x0
        x0 = x0 + ks[(i + 1) % 3]
        x1 = x1 + ks[(i + 2) % 3] + np.uint32(i + 1)
    return x0 ^ x1


def _gumbel(idx_u32):
    bits = _gumbel_bits(idx_u32)
    mant = jax.lax.shift_right_logical(bits, np.uint32(9)) | np.uint32(0x3F800000)
    f = jax.lax.bitcast_convert_type(mant, jnp.float32) - np.float32(1.0)
    u = jnp.maximum(_TINY, f + _TINY)
    return -jnp.log(-jnp.log(u))


def _policy_kernel(x_ref, w_ref, b_ref, out_ref, best_v, best_i, *, num_blocks):
    blk = pl.program_id(0)
    B = x_ref.shape[0]
    T = w_ref.shape[0]

    @pl.when(blk == 0)
    def _init():
        best_v[...] = jnp.full((B, 1), -jnp.inf, jnp.float32)
        best_i[...] = jnp.zeros((B, 1), jnp.int32)

    logits = jax.lax.dot_general(
        x_ref[...],
        w_ref[...],
        (((1,), (1,)), ((), ())),
        preferred_element_type=jnp.float32,
    )
    logits = logits + b_ref[...]

    col = jax.lax.broadcasted_iota(jnp.int32, (B, T), 1) + blk * T
    row = jax.lax.broadcasted_iota(jnp.int32, (B, T), 0)
    flat = (row * _A + col).astype(jnp.uint32)
    cand = logits + _gumbel(flat)
    cand = jnp.where(col < _A, cand, -jnp.inf)

    m = jnp.max(cand, axis=1, keepdims=True)
    idx = jnp.min(
        jnp.where(cand == m, col, jnp.int32(0x7FFFFFFF)), axis=1, keepdims=True
    )
    better = m > best_v[...]
    best_v[...] = jnp.where(better, m, best_v[...])
    best_i[...] = jnp.where(better, idx, best_i[...])

    @pl.when(blk == num_blocks - 1)
    def _write():
        out_ref[...] = best_i[...]


def kernel(x, W, b):
    B, D = x.shape
    A = W.shape[0]
    G = pl.cdiv(A, _TILE)
    sample = pl.pallas_call(
        functools.partial(_policy_kernel, num_blocks=G),
        grid=(G,),
        in_specs=[
            pl.BlockSpec((B, D), lambda i: (0, 0)),
            pl.BlockSpec((_TILE, D), lambda i: (i, 0)),
            pl.BlockSpec((1, _TILE), lambda i: (0, i)),
        ],
        out_specs=pl.BlockSpec((B, 1), lambda i: (0, 0)),
        out_shape=jax.ShapeDtypeStruct((B, 1), jnp.int32),
        scratch_shapes=[
            pltpu.VMEM((B, 1), jnp.float32),
            pltpu.VMEM((B, 1), jnp.int32),
        ],
    )(x, W, b.reshape(1, A))
    return sample.astype(jnp.int64)
